# baseline (speedup 1.0000x reference)
import jax
import jax.numpy as jnp
from jax import lax
from jax.experimental import pallas as pl
from jax.experimental.pallas import tpu as pltpu
from jax.experimental.pallas import tpu_sc as plsc
from jax.experimental.layout import Layout, with_layout_constraint

L = 16
NW = 32
B = 16384
N = 1000000
NBLK = 7816
NFW = NBLK * 512
BW = B // NW
ICH = 128
NCH = BW // ICH


def _body(ladj_hbm, aadj_hbm, lloc_hbm, aloc_hbm, ttc_hbm, cam_hbm, sen_hbm,
          out_hbm, tout_hbm,
          camv, w0, w1, w2, senv, gat, loc, outv, outt, ttcv, sem0, sem1):
    wid = lax.axis_index("s") * 2 + lax.axis_index("c")
    base = wid * BW

    pltpu.sync_copy(cam_hbm.at[pl.ds(wid * NCH, NCH)], camv)
    lane = lax.iota(jnp.int32, L)

    def build(g, carry):
        j = g // (ICH // L)
        sl = pl.ds((g % (ICH // L)) * L, L)
        cam = camv[j, sl]
        w = 512 * (cam >> 7) + (cam & 127)
        w0[j, sl] = w
        w1[j, sl] = w + 128
        w2[j, sl] = w + 256
        return carry

    lax.fori_loop(0, NCH * (ICH // L), build, 0)

    idxs = (w0, w1, w2)
    copies = []
    for j in range(NCH):
        dst = pl.ds(j * ICH, ICH)
        for c in range(3):
            copies.append(pltpu.async_copy(
                ladj_hbm.at[idxs[c].at[j]], gat.at[c].at[dst], sem0))
            copies.append(pltpu.async_copy(
                aadj_hbm.at[idxs[c].at[j]], gat.at[c + 3].at[dst], sem1))

    pltpu.sync_copy(sen_hbm.at[pl.ds(base, BW)], senv)
    for c in range(3):
        pltpu.sync_copy(lloc_hbm.at[pl.ds(c * B + base, BW)], loc.at[c])
        pltpu.sync_copy(aloc_hbm.at[pl.ds(c * B + base, BW)], loc.at[c + 3])
    pltpu.sync_copy(ttc_hbm, ttcv)

    def ttc_group(g, carry):
        sidx = senv[pl.ds(g * L, L)]
        outt[pl.ds(g * L, L)] = plsc.load_gather(ttcv, [sidx])
        return carry

    lax.fori_loop(0, BW // L, ttc_group, 0)

    for cp in copies:
        cp.wait()

    def combine(g, carry):
        sl = pl.ds(g * L, L)
        for c in range(6):
            outv[c, sl] = gat[c, sl] + loc[c, sl]
        return carry

    lax.fori_loop(0, BW // L, combine, 0)

    for c in range(6):
        pltpu.sync_copy(outv.at[c], out_hbm.at[c].at[pl.ds(base, BW)])
    pltpu.sync_copy(outt, tout_hbm.at[pl.ds(base, BW)])


def _flat_view(table):
    y = jnp.pad(table, ((0, NBLK * 128 - N), (0, 1)))
    y = y.reshape(NBLK, 128, 4)
    y = with_layout_constraint(y, Layout(major_to_minor=(0, 2, 1)))
    return y.transpose(0, 2, 1).reshape(-1)


@jax.jit
def kernel(linear_velocities_local, angular_velocities_local,
           linear_velocity_adjustment, angular_velocity_adjustment,
           time_to_center_pixel_adjustment, cam_idx, sensor_idx):
    run = pl.kernel(
        _body,
        out_type=(
            jax.ShapeDtypeStruct((6, B), jnp.float32),
            jax.ShapeDtypeStruct((B,), jnp.float32),
        ),
        mesh=plsc.VectorSubcoreMesh(core_axis_name="c", subcore_axis_name="s"),
        compiler_params=pltpu.CompilerParams(
            needs_layout_passes=False, use_tc_tiling_on_sc=False),
        scratch_types=[
            pltpu.VMEM((NCH, ICH), jnp.int32),
            pltpu.VMEM((NCH, ICH), jnp.int32),
            pltpu.VMEM((NCH, ICH), jnp.int32),
            pltpu.VMEM((NCH, ICH), jnp.int32),
            pltpu.VMEM((BW,), jnp.int32),
            pltpu.VMEM((6, BW), jnp.float32),
            pltpu.VMEM((6, BW), jnp.float32),
            pltpu.VMEM((6, BW), jnp.float32),
            pltpu.VMEM((BW,), jnp.float32),
            pltpu.VMEM((26,), jnp.float32),
            pltpu.SemaphoreType.DMA,
            pltpu.SemaphoreType.DMA,
        ],
    )
    out_t, ttc = run(
        _flat_view(linear_velocity_adjustment),
        _flat_view(angular_velocity_adjustment),
        linear_velocities_local.T.reshape(-1),
        angular_velocities_local.T.reshape(-1),
        time_to_center_pixel_adjustment,
        cam_idx.astype(jnp.int32).reshape(B // ICH, ICH),
        sensor_idx.astype(jnp.int32),
    )
    return out_t.T, ttc


# speedup vs baseline: 1.0004x; 1.0004x over previous
import jax
import jax.numpy as jnp
from jax import lax
from jax.experimental import pallas as pl
from jax.experimental.pallas import tpu as pltpu
from jax.experimental.pallas import tpu_sc as plsc
from jax.experimental.layout import Layout, with_layout_constraint

L = 16
NW = 32
B = 16384
N = 1000000
NBLK = 7816
NFW = NBLK * 512
BW = B // NW
ICH = 128
NCH = BW // ICH


def _body(ladj_hbm, aadj_hbm, lloc_hbm, aloc_hbm, ttc_hbm, cam_hbm, sen_hbm,
          out_hbm, tout_hbm,
          camv, w0, w1, w2, senv, gat, loc, outv, outt, ttcv, sem0, sem1):
    wid = lax.axis_index("s") * 2 + lax.axis_index("c")
    base = wid * BW

    pltpu.sync_copy(cam_hbm.at[pl.ds(wid * NCH, NCH)], camv)

    def build(g, carry):
        j = g // (ICH // L)
        sl = pl.ds((g % (ICH // L)) * L, L)
        cam = camv[j, sl]
        w = 512 * (cam >> 7) + (cam & 127)
        w0[j, sl] = w
        w1[j, sl] = w + 128
        w2[j, sl] = w + 256
        return carry

    lax.fori_loop(0, NCH * (ICH // L), build, 0)

    idxs = (w0, w1, w2)
    copies = []
    for j in range(NCH):
        dst = pl.ds(j * ICH, ICH)
        for c in range(3):
            copies.append(pltpu.async_copy(
                ladj_hbm.at[idxs[c].at[j]], gat.at[c].at[dst], sem0))
            copies.append(pltpu.async_copy(
                aadj_hbm.at[idxs[c].at[j]], gat.at[c + 3].at[dst], sem1))

    pltpu.sync_copy(sen_hbm.at[pl.ds(base, BW)], senv)
    for c in range(3):
        pltpu.sync_copy(lloc_hbm.at[pl.ds(c * B + base, BW)], loc.at[c])
        pltpu.sync_copy(aloc_hbm.at[pl.ds(c * B + base, BW)], loc.at[c + 3])
    pltpu.sync_copy(ttc_hbm, ttcv)

    def ttc_group(g, carry):
        sidx = senv[pl.ds(g * L, L)]
        outt[pl.ds(g * L, L)] = plsc.load_gather(ttcv, [sidx])
        return carry

    lax.fori_loop(0, BW // L, ttc_group, 0)

    for cp in copies:
        cp.wait()

    def combine(g, carry):
        sl = pl.ds(g * L, L)
        for c in range(6):
            outv[c, sl] = gat[c, sl] + loc[c, sl]
        return carry

    lax.fori_loop(0, BW // L, combine, 0)

    for c in range(6):
        pltpu.sync_copy(outv.at[c], out_hbm.at[c].at[pl.ds(base, BW)])
    pltpu.sync_copy(outt, tout_hbm.at[pl.ds(base, BW)])


def _flat_view(table):
    y = jnp.pad(table, ((0, NBLK * 128 - N), (0, 1)))
    y = y.reshape(NBLK, 128, 4)
    y = with_layout_constraint(y, Layout(major_to_minor=(0, 2, 1)))
    return y.transpose(0, 2, 1).reshape(-1)


@jax.jit
def kernel(linear_velocities_local, angular_velocities_local,
           linear_velocity_adjustment, angular_velocity_adjustment,
           time_to_center_pixel_adjustment, cam_idx, sensor_idx):
    run = pl.kernel(
        _body,
        out_type=(
            jax.ShapeDtypeStruct((6, B), jnp.float32),
            jax.ShapeDtypeStruct((B,), jnp.float32),
        ),
        mesh=plsc.VectorSubcoreMesh(core_axis_name="c", subcore_axis_name="s"),
        compiler_params=pltpu.CompilerParams(
            needs_layout_passes=False, use_tc_tiling_on_sc=False),
        scratch_types=[
            pltpu.VMEM((NCH, ICH), jnp.int32),
            pltpu.VMEM((NCH, ICH), jnp.int32),
            pltpu.VMEM((NCH, ICH), jnp.int32),
            pltpu.VMEM((NCH, ICH), jnp.int32),
            pltpu.VMEM((BW,), jnp.int32),
            pltpu.VMEM((6, BW), jnp.float32),
            pltpu.VMEM((6, BW), jnp.float32),
            pltpu.VMEM((6, BW), jnp.float32),
            pltpu.VMEM((BW,), jnp.float32),
            pltpu.VMEM((26,), jnp.float32),
            pltpu.SemaphoreType.DMA,
            pltpu.SemaphoreType.DMA,
        ],
    )
    out_t, ttc = run(
        _flat_view(linear_velocity_adjustment),
        _flat_view(angular_velocity_adjustment),
        linear_velocities_local.T.reshape(-1),
        angular_velocities_local.T.reshape(-1),
        time_to_center_pixel_adjustment,
        cam_idx.astype(jnp.int32).reshape(B // ICH, ICH),
        sensor_idx.astype(jnp.int32),
    )
    return out_t.T, ttc
